# baseline (speedup 1.0000x reference)
import functools

import jax
import jax.numpy as jnp
from jax import lax
from jax.experimental import pallas as pl
from jax.experimental.pallas import tpu as pltpu
from jax.experimental.pallas import tpu_sc as plsc

_INFO = plsc.get_sparse_core_info()
_NC = _INFO.num_cores
_NS = _INFO.num_subcores
_NW = _NC * _NS
_CHUNK = 128


def _sc_gather(uid3, mid3, user_table, movie_table, ch, emb):
    mesh = plsc.VectorSubcoreMesh(core_axis_name="c", subcore_axis_name="s")
    out_sds = jax.ShapeDtypeStruct((_NW, ch, _CHUNK, emb), jnp.float32)

    @functools.partial(
        pl.kernel,
        mesh=mesh,
        out_type=[out_sds, out_sds],
        compiler_params=pltpu.CompilerParams(use_tc_tiling_on_sc=False),
        scratch_types=[
            pltpu.VMEM((ch, _CHUNK), jnp.int32),
            pltpu.VMEM((ch, _CHUNK), jnp.int32),
            pltpu.VMEM((ch, _CHUNK, emb), jnp.float32),
            pltpu.VMEM((ch, _CHUNK, emb), jnp.float32),
            pltpu.SemaphoreType.DMA,
        ],
    )
    def gather_kernel(uid_hbm, mid_hbm, utab_hbm, mtab_hbm,
                      uout_hbm, mout_hbm,
                      uidx_v, midx_v, urows_v, mrows_v, sem):
        wid = lax.axis_index("s") * _NC + lax.axis_index("c")
        pltpu.sync_copy(uid_hbm.at[wid], uidx_v)
        pltpu.sync_copy(mid_hbm.at[wid], midx_v)
        copies = []
        for j in range(ch):
            copies.append(
                pltpu.async_copy(utab_hbm.at[uidx_v.at[j]], urows_v.at[j], sem))
            copies.append(
                pltpu.async_copy(mtab_hbm.at[midx_v.at[j]], mrows_v.at[j], sem))
        for c in copies:
            c.wait()
        pltpu.sync_copy(urows_v, uout_hbm.at[wid])
        pltpu.sync_copy(mrows_v, mout_hbm.at[wid])

    return gather_kernel(uid3, mid3, user_table, movie_table)


def _mlp_body(u_ref, m_ref, w1a_ref, w1b_ref, b1_ref, w2_ref, b2_ref,
              w3t_ref, b3_ref, out_ref):
    h1 = jnp.dot(u_ref[:, :], w1a_ref[:, :], preferred_element_type=jnp.float32)
    h1 = h1 + jnp.dot(m_ref[:, :], w1b_ref[:, :],
                      preferred_element_type=jnp.float32)
    h1 = jnp.maximum(h1 + b1_ref[:, :], 0.0)
    h2 = jnp.dot(h1, w2_ref[:, :], preferred_element_type=jnp.float32)
    h2 = jnp.maximum(h2 + b2_ref[:, :], 0.0)
    out = jnp.sum(h2 * w3t_ref[:, :], axis=1, keepdims=True) + b3_ref[0, 0]
    out_ref[:, :] = out


def _tc_mlp(u_emb, m_emb, W1, b1, W2, b2, W3, b3, bm):
    B, emb = u_emb.shape
    h1d = W1.shape[1]
    h2d = W2.shape[1]
    grid = (B // bm,)
    return pl.pallas_call(
        _mlp_body,
        grid=grid,
        in_specs=[
            pl.BlockSpec((bm, emb), lambda i: (i, 0)),
            pl.BlockSpec((bm, emb), lambda i: (i, 0)),
            pl.BlockSpec((emb, h1d), lambda i: (0, 0)),
            pl.BlockSpec((emb, h1d), lambda i: (0, 0)),
            pl.BlockSpec((1, h1d), lambda i: (0, 0)),
            pl.BlockSpec((h1d, h2d), lambda i: (0, 0)),
            pl.BlockSpec((1, h2d), lambda i: (0, 0)),
            pl.BlockSpec((1, h2d), lambda i: (0, 0)),
            pl.BlockSpec((1, 1), lambda i: (0, 0)),
        ],
        out_specs=pl.BlockSpec((bm, 1), lambda i: (i, 0)),
        out_shape=jax.ShapeDtypeStruct((B, 1), jnp.float32),
        compiler_params=pltpu.CompilerParams(
            dimension_semantics=("arbitrary",)),
    )(u_emb, m_emb, W1[:emb], W1[emb:], b1.reshape(1, h1d), W2,
      b2.reshape(1, h2d), W3.reshape(1, h2d), b3.reshape(1, 1))


def kernel(user_id, movie_title, user_table, movie_table,
           W1, b1, W2, b2, W3, b3):
    B = user_id.shape[0]
    emb = user_table.shape[1]
    ch = B // (_NW * _CHUNK)
    uid3 = user_id.reshape(_NW, ch, _CHUNK)
    mid3 = movie_title.reshape(_NW, ch, _CHUNK)
    u4, m4 = _sc_gather(uid3, mid3, user_table, movie_table, ch, emb)
    u_emb = u4.reshape(B, emb)
    m_emb = m4.reshape(B, emb)
    return _tc_mlp(u_emb, m_emb, W1, b1, W2, b2, W3, b3, bm=2048)


# speedup vs baseline: 1.0036x; 1.0036x over previous
import functools

import jax
import jax.numpy as jnp
from jax import lax
from jax.experimental import pallas as pl
from jax.experimental.pallas import tpu as pltpu
from jax.experimental.pallas import tpu_sc as plsc

_INFO = plsc.get_sparse_core_info()
_NC = _INFO.num_cores
_NS = _INFO.num_subcores
_NW = _NC * _NS
_CW = 128


def _sc_gather(up3, mp3, u2, m2, nchunks):
    B = _NW * nchunks * _CW
    mesh = plsc.VectorSubcoreMesh(core_axis_name="c", subcore_axis_name="s")
    out_sds = jax.ShapeDtypeStruct((B, 128), jnp.float32)

    @functools.partial(
        pl.kernel,
        mesh=mesh,
        out_type=[out_sds, out_sds],
        scratch_types=[
            pltpu.VMEM((nchunks, _CW), jnp.int32),
            pltpu.VMEM((nchunks, _CW), jnp.int32),
            pltpu.VMEM((2, _CW, 128), jnp.float32),
            pltpu.VMEM((2, _CW, 128), jnp.float32),
            pltpu.SemaphoreType.DMA,
        ],
    )
    def gather_kernel(up_hbm, mp_hbm, u2_hbm, m2_hbm,
                      uout_hbm, mout_hbm,
                      uidx, midx, ubuf, mbuf, sem):
        wid = lax.axis_index("s") * _NC + lax.axis_index("c")
        pltpu.sync_copy(up_hbm.at[wid], uidx)
        pltpu.sync_copy(mp_hbm.at[wid], midx)

        def fire(c):
            s = c % 2
            return (
                pltpu.async_copy(u2_hbm.at[uidx.at[c]], ubuf.at[s], sem),
                pltpu.async_copy(m2_hbm.at[midx.at[c]], mbuf.at[s], sem),
            )

        pend = [fire(0), fire(1)]
        for c in range(nchunks):
            cu, cm = pend[c]
            cu.wait()
            cm.wait()
            s = c % 2
            base = (wid * nchunks + c) * _CW
            pltpu.sync_copy(ubuf.at[s], uout_hbm.at[pl.ds(base, _CW)])
            pltpu.sync_copy(mbuf.at[s], mout_hbm.at[pl.ds(base, _CW)])
            if c + 2 < nchunks:
                pend.append(fire(c + 2))

    return gather_kernel(up3, mp3, u2, m2)


def _mlp_body(u_ref, m_ref, hu_ref, hm_ref, w1a_ref, w1b_ref, b1_ref,
              w2_ref, b2_ref, w3t_ref, b3_ref, out_ref):
    bm = u_ref.shape[0]
    left = (lax.broadcasted_iota(jnp.int32, (bm, 128), 1) < 64).astype(
        jnp.float32)
    mask_u = left + hu_ref[:, :] * (1.0 - 2.0 * left)
    mask_m = left + hm_ref[:, :] * (1.0 - 2.0 * left)
    h1 = jnp.dot(u_ref[:, :] * mask_u, w1a_ref[:, :],
                 preferred_element_type=jnp.float32)
    h1 = h1 + jnp.dot(m_ref[:, :] * mask_m, w1b_ref[:, :],
                      preferred_element_type=jnp.float32)
    h1 = jnp.maximum(h1 + b1_ref[:, :], 0.0)
    h2 = jnp.dot(h1, w2_ref[:, :], preferred_element_type=jnp.float32)
    h2 = jnp.maximum(h2 + b2_ref[:, :], 0.0)
    out = jnp.sum(h2 * w3t_ref[:, :], axis=1, keepdims=True) + b3_ref[0, 0]
    out_ref[:, :] = out


def _tc_mlp(u128, m128, hu, hm, W1, b1, W2, b2, W3, b3, bm):
    B = u128.shape[0]
    emb = W1.shape[0] // 2
    h1d = W1.shape[1]
    h2d = W2.shape[1]
    w1a2 = jnp.concatenate([W1[:emb], W1[:emb]], axis=0)
    w1b2 = jnp.concatenate([W1[emb:], W1[emb:]], axis=0)
    grid = (B // bm,)
    return pl.pallas_call(
        _mlp_body,
        grid=grid,
        in_specs=[
            pl.BlockSpec((bm, 128), lambda i: (i, 0)),
            pl.BlockSpec((bm, 128), lambda i: (i, 0)),
            pl.BlockSpec((bm, 1), lambda i: (i, 0)),
            pl.BlockSpec((bm, 1), lambda i: (i, 0)),
            pl.BlockSpec((128, h1d), lambda i: (0, 0)),
            pl.BlockSpec((128, h1d), lambda i: (0, 0)),
            pl.BlockSpec((1, h1d), lambda i: (0, 0)),
            pl.BlockSpec((h1d, h2d), lambda i: (0, 0)),
            pl.BlockSpec((1, h2d), lambda i: (0, 0)),
            pl.BlockSpec((1, h2d), lambda i: (0, 0)),
            pl.BlockSpec((1, 1), lambda i: (0, 0)),
        ],
        out_specs=pl.BlockSpec((bm, 1), lambda i: (i, 0)),
        out_shape=jax.ShapeDtypeStruct((B, 1), jnp.float32),
        compiler_params=pltpu.CompilerParams(
            dimension_semantics=("arbitrary",)),
    )(u128, m128, hu, hm, w1a2, w1b2, b1.reshape(1, h1d), W2,
      b2.reshape(1, h2d), W3.reshape(1, h2d), b3.reshape(1, 1))


def kernel(user_id, movie_title, user_table, movie_table,
           W1, b1, W2, b2, W3, b3):
    B = user_id.shape[0]
    nchunks = B // (_NW * _CW)
    up3 = (user_id >> 1).reshape(_NW, nchunks, _CW)
    mp3 = (movie_title >> 1).reshape(_NW, nchunks, _CW)
    hu = (user_id & 1).astype(jnp.float32).reshape(B, 1)
    hm = (movie_title & 1).astype(jnp.float32).reshape(B, 1)
    nu = (user_table.shape[0] - 1) // 2
    nm = (movie_table.shape[0] - 1) // 2
    u2 = user_table[:2 * nu].reshape(nu, 128)
    m2 = movie_table[:2 * nm].reshape(nm, 128)
    u128, m128 = _sc_gather(up3, mp3, u2, m2, nchunks)
    return _tc_mlp(u128, m128, hu, hm, W1, b1, W2, b2, W3, b3, bm=2048)
